# Initial kernel scaffold; baseline (speedup 1.0000x reference)
import jax
import jax.numpy as jnp
from jax.experimental import pallas as pl


def kernel(x, edge_index, W1, b1, W2, b2, W3, b3, Wp1, bp1, Wp2, bp2, Wt1, bt1, Wt2, bt2):
    raise NotImplementedError("write your pallas kernel here")


# speedup vs baseline: 10.3848x; 10.3848x over previous
import functools
import math

import jax
import jax.numpy as jnp
from jax import lax
from jax.experimental import pallas as pl
from jax.experimental.pallas import tpu as pltpu
from jax.experimental.pallas import tpu_sc as plsc

N_NODES = 10000
N_PAD = 10240
D = 128
NC = 2
NS = 16
NW = NC * NS
CHUNK = 128
ROWS_PER_TILE = N_PAD // NS
RBLK = 512


def _sc_mesh():
    return plsc.VectorSubcoreMesh(core_axis_name="c", subcore_axis_name="s")


@functools.partial(jax.jit, static_argnums=(1,))
def _sc_degree(dst_blocks, ept):
    @functools.partial(
        pl.kernel,
        mesh=_sc_mesh(),
        out_type=jax.ShapeDtypeStruct((NW, N_PAD), jnp.float32),
        scratch_types=[
            pltpu.VMEM((ept, CHUNK), jnp.int32),
            pltpu.VMEM((N_PAD,), jnp.float32),
        ],
        compiler_params=pltpu.CompilerParams(needs_layout_passes=False),
    )
    def deg_kernel(dst_hbm, out_hbm, dst_v, deg_v):
        c = lax.axis_index("c")
        s = lax.axis_index("s")
        wid = s * NC + c
        pltpu.sync_copy(dst_hbm.at[wid], dst_v)

        zeros16 = jnp.zeros((16,), jnp.float32)

        def zero_body(i, _):
            deg_v[pl.ds(i * 16, 16)] = zeros16
            return 0

        lax.fori_loop(0, N_PAD // 16, zero_body, 0)

        ones16 = jnp.full((16,), 1.0, jnp.float32)

        def count_body(j, _):
            for k in range(CHUNK // 16):
                idx = dst_v[j, pl.ds(k * 16, 16)]
                plsc.addupdate_scatter(deg_v, [idx], ones16)
            return 0

        lax.fori_loop(0, ept, count_body, 0)
        pltpu.sync_copy(deg_v, out_hbm.at[wid])

    return deg_kernel(dst_blocks)


@functools.partial(jax.jit, static_argnums=(3,))
def _sc_scatter(src_blocks, dst_blocks, g, ept):
    @functools.partial(
        pl.kernel,
        mesh=_sc_mesh(),
        out_type=jax.ShapeDtypeStruct((NC, N_PAD, D), jnp.float32),
        scratch_types=[
            pltpu.VMEM((ept, CHUNK), jnp.int32),
            pltpu.VMEM((ept, CHUNK), jnp.int32),
            pltpu.VMEM((CHUNK, D), jnp.float32),
            pltpu.VMEM_SHARED((N_PAD, D), jnp.float32),
            pltpu.SemaphoreType.DMA,
        ],
        compiler_params=pltpu.CompilerParams(needs_layout_passes=False),
    )
    def scat_kernel(src_hbm, dst_hbm, g_hbm, out_hbm,
                    src_v, dst_v, rows_v, acc_sh, sem):
        c = lax.axis_index("c")
        s = lax.axis_index("s")
        wid = s * NC + c
        pltpu.sync_copy(src_hbm.at[wid], src_v)
        pltpu.sync_copy(dst_hbm.at[wid], dst_v)
        row0 = s * ROWS_PER_TILE
        pltpu.sync_copy(g_hbm.at[pl.ds(row0, ROWS_PER_TILE)],
                        acc_sh.at[pl.ds(row0, ROWS_PER_TILE)])
        plsc.subcore_barrier()

        def body(j, _):
            cp = pltpu.async_copy(g_hbm.at[src_v.at[j]], rows_v, sem)
            cp.wait()
            pltpu.sync_copy(rows_v, acc_sh.at[dst_v.at[j]], add=True)
            return 0

        lax.fori_loop(0, ept, body, 0)
        plsc.subcore_barrier()
        pltpu.sync_copy(acc_sh.at[pl.ds(row0, ROWS_PER_TILE)],
                        out_hbm.at[c].at[pl.ds(row0, ROWS_PER_TILE)])

    return scat_kernel(src_blocks, dst_blocks, g)


def _tc_stage0(deg_partials, x_pad, W1):
    def body(deg_ref, x_ref, w_ref, g_ref, dinv_ref):
        i = pl.program_id(0)
        deg = jnp.sum(deg_ref[...], axis=0) + 1.0
        rows = i * RBLK + lax.broadcasted_iota(jnp.int32, (RBLK, 1), 0)
        dinv_col = jnp.where(rows < N_NODES, lax.rsqrt(deg)[:, None], 0.0)
        dmat = jnp.broadcast_to(dinv_col, (RBLK, D))
        g = jnp.dot(x_ref[...], w_ref[...],
                    preferred_element_type=jnp.float32) * dmat
        g_ref[...] = g
        dinv_ref[...] = dmat

    grid = (N_PAD // RBLK,)
    return pl.pallas_call(
        body,
        grid=grid,
        in_specs=[
            pl.BlockSpec((NW, RBLK), lambda i: (0, i)),
            pl.BlockSpec((RBLK, D), lambda i: (i, 0)),
            pl.BlockSpec((D, D), lambda i: (0, 0)),
        ],
        out_specs=[
            pl.BlockSpec((RBLK, D), lambda i: (i, 0)),
            pl.BlockSpec((RBLK, D), lambda i: (i, 0)),
        ],
        out_shape=[
            jax.ShapeDtypeStruct((N_PAD, D), jnp.float32),
            jax.ShapeDtypeStruct((N_PAD, D), jnp.float32),
        ],
    )(deg_partials, x_pad, W1)


def _tc_stage_mid(S, g_prev, dinv2d, b, W_next):
    def body(s_ref, g_ref, d_ref, b_ref, w_ref, out_ref):
        sarr = s_ref[...]
        agg = sarr[0] + sarr[1] - g_ref[...]
        d = d_ref[...]
        u = jax.nn.relu(agg * d + b_ref[...])
        out_ref[...] = jnp.dot(u, w_ref[...],
                               preferred_element_type=jnp.float32) * d

    grid = (N_PAD // RBLK,)
    return pl.pallas_call(
        body,
        grid=grid,
        in_specs=[
            pl.BlockSpec((NC, RBLK, D), lambda i: (0, i, 0)),
            pl.BlockSpec((RBLK, D), lambda i: (i, 0)),
            pl.BlockSpec((RBLK, D), lambda i: (i, 0)),
            pl.BlockSpec((1, D), lambda i: (0, 0)),
            pl.BlockSpec((D, D), lambda i: (0, 0)),
        ],
        out_specs=pl.BlockSpec((RBLK, D), lambda i: (i, 0)),
        out_shape=jax.ShapeDtypeStruct((N_PAD, D), jnp.float32),
    )(S, g_prev, dinv2d, b, W_next)


def _tc_stage_final(S, g_prev, dinv2d, b3, Wp1, bp1, Wt1, bt1,
                    Wp2pad, Wt2pad, bcat):
    def body(s_ref, g_ref, d_ref, b_ref, wp1_ref, bp1_ref, wt1_ref, bt1_ref,
             wp2_ref, wt2_ref, bc_ref, out_ref):
        sarr = s_ref[...]
        agg = sarr[0] + sarr[1] - g_ref[...]
        u = jax.nn.relu(agg * d_ref[...] + b_ref[...])
        ph = jax.nn.relu(jnp.dot(u, wp1_ref[...],
                                 preferred_element_type=jnp.float32)
                         + bp1_ref[...])
        th = jax.nn.relu(jnp.dot(u, wt1_ref[...],
                                 preferred_element_type=jnp.float32)
                         + bt1_ref[...])
        out_ref[...] = (jnp.dot(ph, wp2_ref[...],
                                preferred_element_type=jnp.float32)
                        + jnp.dot(th, wt2_ref[...],
                                  preferred_element_type=jnp.float32)
                        + bc_ref[...])

    grid = (N_PAD // RBLK,)
    H2 = Wt1.shape[1]
    return pl.pallas_call(
        body,
        grid=grid,
        in_specs=[
            pl.BlockSpec((NC, RBLK, D), lambda i: (0, i, 0)),
            pl.BlockSpec((RBLK, D), lambda i: (i, 0)),
            pl.BlockSpec((RBLK, D), lambda i: (i, 0)),
            pl.BlockSpec((1, D), lambda i: (0, 0)),
            pl.BlockSpec((D, D), lambda i: (0, 0)),
            pl.BlockSpec((1, D), lambda i: (0, 0)),
            pl.BlockSpec((D, H2), lambda i: (0, 0)),
            pl.BlockSpec((1, H2), lambda i: (0, 0)),
            pl.BlockSpec((D, D), lambda i: (0, 0)),
            pl.BlockSpec((H2, D), lambda i: (0, 0)),
            pl.BlockSpec((1, D), lambda i: (0, 0)),
        ],
        out_specs=pl.BlockSpec((RBLK, D), lambda i: (i, 0)),
        out_shape=jax.ShapeDtypeStruct((N_PAD, D), jnp.float32),
    )(S, g_prev, dinv2d, b3, Wp1, bp1, Wt1, bt1, Wp2pad, Wt2pad, bcat)


def kernel(x, edge_index, W1, b1, W2, b2, W3, b3,
           Wp1, bp1, Wp2, bp2, Wt1, bt1, Wt2, bt2):
    E = edge_index.shape[1]
    ept = math.ceil(E / (NW * CHUNK))
    e_tot = NW * CHUNK * ept
    pad_e = e_tot - E

    pad_idx = jnp.full((pad_e,), N_NODES, jnp.int32)
    src = jnp.concatenate([edge_index[0].astype(jnp.int32), pad_idx])
    dst = jnp.concatenate([edge_index[1].astype(jnp.int32), pad_idx])
    src_blocks = src.reshape(NW, ept, CHUNK)
    dst_blocks = dst.reshape(NW, ept, CHUNK)

    x_pad = jnp.pad(x, ((0, N_PAD - x.shape[0]), (0, 0)))

    H2 = Wt1.shape[1]
    Wp2pad = jnp.zeros((D, D), jnp.float32).at[:, 0:2].set(Wp2)
    Wt2pad = jnp.zeros((H2, D), jnp.float32).at[:, 2:3].set(Wt2)
    bcat = jnp.zeros((1, D), jnp.float32).at[0, 0:2].set(bp2).at[0, 2].set(bt2[0])

    deg_partials = _sc_degree(dst_blocks, ept)
    g1, dinv2d = _tc_stage0(deg_partials, x_pad, W1)

    S1 = _sc_scatter(src_blocks, dst_blocks, g1, ept)
    g2 = _tc_stage_mid(S1, g1, dinv2d, b1.reshape(1, D), W2)

    S2 = _sc_scatter(src_blocks, dst_blocks, g2, ept)
    g3 = _tc_stage_mid(S2, g2, dinv2d, b2.reshape(1, D), W3)

    S3 = _sc_scatter(src_blocks, dst_blocks, g3, ept)
    out_pad = _tc_stage_final(S3, g3, dinv2d, b3.reshape(1, D),
                              Wp1, bp1.reshape(1, D), Wt1, bt1.reshape(1, H2),
                              Wp2pad, Wt2pad, bcat)
    return out_pad[:N_NODES, :3]
